# baseline (speedup 1.0000x reference)
import functools

import jax
import jax.numpy as jnp
from jax.experimental import pallas as pl
from jax.experimental.pallas import tpu as pltpu

_K = 256
_SROWS = 64
_DROWS = 8


def _sortable_i32(x):
    b = jax.lax.bitcast_convert_type(x, jnp.int32)
    return b ^ ((b >> 31) & jnp.int32(0x7FFFFFFF))


def _cellmax(x):
    v = x
    while v.shape[1] > 256:
        h = v.shape[1] // 2
        v = jnp.maximum(v[:, :h], v[:, h:])
    return v


def _loser(s):
    return jnp.sign(s) * (1.0 - s)


def _sel_body(s_ref, t_ref):
    s = s_ref[...]
    rows, n = s.shape
    l = _loser(s)
    kw = _sortable_i32(s)
    kl = _sortable_i32(l)
    kk = jnp.int32(_K)

    cmw = _cellmax(s)
    cml = _cellmax(l)
    lbw = _sortable_i32(jnp.min(cmw, axis=1, keepdims=True))
    lbl = _sortable_i32(jnp.min(cml, axis=1, keepdims=True))
    ubw = _sortable_i32(jnp.max(cmw, axis=1, keepdims=True)) + 1
    ubl = _sortable_i32(jnp.max(cml, axis=1, keepdims=True)) + 1

    cw_lo0 = jnp.sum((kw >= lbw).astype(jnp.int32), axis=1, keepdims=True)
    cl_lo0 = jnp.sum((kl >= lbl).astype(jnp.int32), axis=1, keepdims=True)
    zero = jnp.zeros((rows, 1), jnp.int32)

    def vcond(carry):
        lw, hw, ll, hl = carry[0], carry[1], carry[2], carry[3]
        return jnp.any(hw > lw + 1) | jnp.any(hl > ll + 1)

    def vstep(carry):
        lw, hw, ll, hl, cwlo, cwhi, cllo, clhi = carry
        mw = (lw >> 1) + (hw >> 1) + (lw & hw & 1)
        ml2 = (ll >> 1) + (hl >> 1) + (ll & hl & 1)
        cwn = jnp.sum((kw >= mw).astype(jnp.int32), axis=1, keepdims=True)
        cln = jnp.sum((kl >= ml2).astype(jnp.int32), axis=1, keepdims=True)
        pw = cwn >= kk
        pl_ = cln >= kk
        dw = hw > lw + 1
        dl = hl > ll + 1
        lw = jnp.where(dw & pw, mw, lw)
        hw = jnp.where(dw & ~pw, mw, hw)
        cwlo = jnp.where(dw & pw, cwn, cwlo)
        cwhi = jnp.where(dw & ~pw, cwn, cwhi)
        ll = jnp.where(dl & pl_, ml2, ll)
        hl = jnp.where(dl & ~pl_, ml2, hl)
        cllo = jnp.where(dl & pl_, cln, cllo)
        clhi = jnp.where(dl & ~pl_, cln, clhi)
        return lw, hw, ll, hl, cwlo, cwhi, cllo, clhi

    lw, _, ll, _, cgew, cgtw, cgel, cgtl = jax.lax.while_loop(
        vcond, vstep,
        (lbw, ubw, lbl, ubl, cw_lo0, zero, cl_lo0, zero))

    rw = kk - cgtw
    rl = kk - cgtl

    idx = jax.lax.broadcasted_iota(jnp.int32, (rows, n), 1)
    eqw = kw == lw
    eql = kl == ll

    needs = jnp.any(cgew > kk) | jnp.any(cgel > kk)

    li0 = jnp.full((rows, 1), -1, jnp.int32)
    hi0i = jnp.full((rows, 1), n - 1, jnp.int32)

    def icond(carry):
        step = carry[4]
        return needs & (step < 15)

    def istep(carry):
        liw, hiw, lil, hil, step = carry
        miw = (liw + hiw) >> 1
        mil = (lil + hil) >> 1
        cwn = jnp.sum((eqw & (idx <= miw)).astype(jnp.int32), axis=1,
                      keepdims=True)
        cln = jnp.sum((eql & (idx <= mil)).astype(jnp.int32), axis=1,
                      keepdims=True)
        pw = cwn >= rw
        pl_ = cln >= rl
        hiw = jnp.where(pw, miw, hiw)
        liw = jnp.where(pw, liw, miw)
        hil = jnp.where(pl_, mil, hil)
        lil = jnp.where(pl_, lil, mil)
        return liw, hiw, lil, hil, step + 1

    _, itw, _, itl, _ = jax.lax.while_loop(
        icond, istep, (li0, hi0i, li0, hi0i, jnp.int32(0)))

    t_ref[...] = jnp.concatenate([lw, itw, ll, itl], axis=1)


def _dense_body(s_ref, t_ref, p_ref, w_ref):
    s = s_ref[...]
    rows, n = s.shape
    thr = t_ref[...]

    m = jnp.max(s, axis=1, keepdims=True)
    e = jnp.exp(s - m)
    z = jnp.sum(e, axis=1, keepdims=True)
    p_ref[...] = e * (1.0 / z)

    l = _loser(s)
    ml_ = jnp.max(l, axis=1, keepdims=True)
    el = jnp.exp(l - ml_)

    kw = _sortable_i32(s)
    kl = _sortable_i32(l)
    vtw = thr[:, 0:1]
    itw = thr[:, 1:2]
    vtl = thr[:, 2:3]
    itl = thr[:, 3:4]

    idx = jax.lax.broadcasted_iota(jnp.int32, (rows, n), 1)
    maskw = (kw > vtw) | ((kw == vtw) & (idx <= itw))
    maskl = (kl > vtl) | ((kl == vtl) & (idx <= itl))

    ew = jnp.where(maskw, e, 0.0)
    zw = jnp.sum(ew, axis=1, keepdims=True)
    w_ref[:, 0:n] = ew * (1.0 / zw)

    elm = jnp.where(maskl, el, 0.0)
    zl = jnp.sum(elm, axis=1, keepdims=True)
    w_ref[:, n:2 * n] = elm * (1.0 / zl)


@jax.jit
def kernel(scores):
    b, n = scores.shape
    thr = pl.pallas_call(
        _sel_body,
        grid=(b // _SROWS,),
        in_specs=[pl.BlockSpec((_SROWS, n), lambda i: (i, 0))],
        out_specs=pl.BlockSpec((_SROWS, 4), lambda i: (i, 0)),
        out_shape=jax.ShapeDtypeStruct((b, 4), jnp.int32),
        compiler_params=pltpu.CompilerParams(
            dimension_semantics=("parallel",),
        ),
    )(scores)
    p_out, w_out = pl.pallas_call(
        _dense_body,
        grid=(b // _DROWS,),
        in_specs=[
            pl.BlockSpec((_DROWS, n), lambda i: (i, 0)),
            pl.BlockSpec((_DROWS, 4), lambda i: (i, 0)),
        ],
        out_specs=[
            pl.BlockSpec((_DROWS, n), lambda i: (i, 0)),
            pl.BlockSpec((_DROWS, 2 * n), lambda i: (i, 0)),
        ],
        out_shape=[
            jax.ShapeDtypeStruct((b, n), jnp.float32),
            jax.ShapeDtypeStruct((b, 2 * n), jnp.float32),
        ],
        compiler_params=pltpu.CompilerParams(
            dimension_semantics=("parallel",),
        ),
    )(scores, thr)
    rho = jnp.full((b,), 0.5, jnp.float32)
    return (w_out, rho, p_out)
